# Initial kernel scaffold; baseline (speedup 1.0000x reference)
import jax
import jax.numpy as jnp
from jax.experimental import pallas as pl


def kernel(x, adj, mask, W_gcn, b_gcn, W_ih, W_hh, b_ih, b_hh, W_fc, b_fc):
    raise NotImplementedError("write your pallas kernel here")


# speedup vs baseline: 1.4492x; 1.4492x over previous
import jax
import jax.numpy as jnp
from jax.experimental import pallas as pl
from jax.experimental.pallas import tpu as pltpu


def _fused_step(x_ref, adj_ref, m_ref, wg_ref, bg_ref, wihT_ref, whhT_ref,
                bih_ref, bhh_ref, wfc_ref, bfc_ref, out_ref, h_ref):
    t = pl.program_id(0)

    @pl.when(t == 0)
    def _init():
        h_ref[...] = jnp.zeros_like(h_ref)

    m = m_ref[0]
    mb = m > 0.5
    A = adj_ref[0].astype(jnp.float32)
    Am = A * m
    ones = jnp.ones((Am.shape[0], 1), dtype=jnp.float32)
    colsum = jax.lax.dot_general(Am, ones, (((0,), (0,)), ((), ())),
                                 preferred_element_type=jnp.float32)
    deg = m * colsum + m
    dinv = jnp.where(mb, jax.lax.rsqrt(jnp.where(mb, deg, 1.0)), 0.0)

    xw = jnp.dot(x_ref[0], wg_ref[...], preferred_element_type=jnp.float32)
    y = xw * dinv
    s = jax.lax.dot_general(Am, y, (((0,), (0,)), ((), ())),
                            preferred_element_type=jnp.float32)
    gcn = jnp.maximum(dinv * (s + y) + bg_ref[...], 0.0)

    h_prev = h_ref[...]
    gi = jnp.dot(gcn, wihT_ref[...], preferred_element_type=jnp.float32)
    gi = gi + bih_ref[...]
    gh = jnp.dot(h_prev, whhT_ref[...], preferred_element_type=jnp.float32)
    gh = gh + bhh_ref[...]
    dh = h_ref.shape[1]
    r = jax.nn.sigmoid(gi[:, :dh] + gh[:, :dh])
    z = jax.nn.sigmoid(gi[:, dh:2 * dh] + gh[:, dh:2 * dh])
    n = jnp.tanh(gi[:, 2 * dh:] + r * gh[:, 2 * dh:])
    h_new = (1.0 - z) * n + z * h_prev
    h = jnp.where(mb, h_new, h_prev)
    h_ref[...] = h

    out_ref[0] = (jnp.dot(h, wfc_ref[...], preferred_element_type=jnp.float32)
                  + bfc_ref[...])


def kernel(x, adj, mask, W_gcn, b_gcn, W_ih, W_hh, b_ih, b_hh, W_fc, b_fc):
    Tn, BN, Din = x.shape
    Bn, _, Nn = mask.shape
    Dh = W_gcn.shape[1]
    Dout = W_fc.shape[1]

    mf = jnp.transpose(mask, (1, 0, 2)).reshape(Tn, BN, 1).astype(jnp.float32)
    seq = pl.pallas_call(
        _fused_step,
        grid=(Tn,),
        in_specs=[
            pl.BlockSpec((1, BN, Din), lambda t: (t, 0, 0)),
            pl.BlockSpec((1, BN, BN), lambda t: (t, 0, 0)),
            pl.BlockSpec((1, BN, 1), lambda t: (t, 0, 0)),
            pl.BlockSpec((Din, Dh), lambda t: (0, 0)),
            pl.BlockSpec((1, Dh), lambda t: (0, 0)),
            pl.BlockSpec((Dh, 3 * Dh), lambda t: (0, 0)),
            pl.BlockSpec((Dh, 3 * Dh), lambda t: (0, 0)),
            pl.BlockSpec((1, 3 * Dh), lambda t: (0, 0)),
            pl.BlockSpec((1, 3 * Dh), lambda t: (0, 0)),
            pl.BlockSpec((Dh, Dout), lambda t: (0, 0)),
            pl.BlockSpec((1, Dout), lambda t: (0, 0)),
        ],
        out_specs=pl.BlockSpec((1, BN, Dout), lambda t: (t, 0, 0)),
        out_shape=jax.ShapeDtypeStruct((Tn, BN, Dout), jnp.float32),
        scratch_shapes=[pltpu.VMEM((BN, Dh), jnp.float32)],
    )(x, adj, mf, W_gcn, b_gcn.reshape(1, Dh), W_ih.T, W_hh.T,
      b_ih.reshape(1, 3 * Dh), b_hh.reshape(1, 3 * Dh), W_fc,
      b_fc.reshape(1, Dout))

    return jnp.transpose(seq.reshape(Tn, Bn, Nn, Dout), (1, 2, 0, 3))


# speedup vs baseline: 1.4953x; 1.0318x over previous
import jax
import jax.numpy as jnp
from jax.experimental import pallas as pl
from jax.experimental.pallas import tpu as pltpu


def _fused_step(x_ref, adj_ref, m_ref, wg_ref, bg_ref, wihT_ref, whhT_ref,
                bih_ref, bhh_ref, wfc_ref, bfc_ref, out_ref, h_ref):
    t = pl.program_id(0)

    @pl.when(t == 0)
    def _init():
        h_ref[...] = jnp.zeros_like(h_ref)

    m = m_ref[0]
    mb = m > 0.5
    A = adj_ref[0].astype(jnp.bfloat16)
    colsum = jax.lax.dot_general(A, m.astype(jnp.bfloat16),
                                 (((0,), (0,)), ((), ())),
                                 preferred_element_type=jnp.float32)
    deg = m * colsum + m
    dinv = jnp.where(mb, jax.lax.rsqrt(jnp.where(mb, deg, 1.0)), 0.0)

    xw = jnp.dot(x_ref[0], wg_ref[...], preferred_element_type=jnp.float32)
    y = xw * dinv
    s = jax.lax.dot_general(A, y.astype(jnp.bfloat16),
                            (((0,), (0,)), ((), ())),
                            preferred_element_type=jnp.float32)
    gcn = jnp.maximum(dinv * (s + y) + bg_ref[...], 0.0)

    h_prev = h_ref[...]
    gi = jnp.dot(gcn, wihT_ref[...], preferred_element_type=jnp.float32)
    gi = gi + bih_ref[...]
    gh = jnp.dot(h_prev, whhT_ref[...], preferred_element_type=jnp.float32)
    gh = gh + bhh_ref[...]
    dh = h_ref.shape[1]
    r = jax.nn.sigmoid(gi[:, :dh] + gh[:, :dh])
    z = jax.nn.sigmoid(gi[:, dh:2 * dh] + gh[:, dh:2 * dh])
    n = jnp.tanh(gi[:, 2 * dh:] + r * gh[:, 2 * dh:])
    h_new = (1.0 - z) * n + z * h_prev
    h = jnp.where(mb, h_new, h_prev)
    h_ref[...] = h

    out_ref[0] = (jnp.dot(h, wfc_ref[...], preferred_element_type=jnp.float32)
                  + bfc_ref[...])


def kernel(x, adj, mask, W_gcn, b_gcn, W_ih, W_hh, b_ih, b_hh, W_fc, b_fc):
    Tn, BN, Din = x.shape
    Bn, _, Nn = mask.shape
    Dh = W_gcn.shape[1]
    Dout = W_fc.shape[1]

    mf = jnp.transpose(mask, (1, 0, 2)).reshape(Tn, BN, 1).astype(jnp.float32)
    seq = pl.pallas_call(
        _fused_step,
        grid=(Tn,),
        in_specs=[
            pl.BlockSpec((1, BN, Din), lambda t: (t, 0, 0)),
            pl.BlockSpec((1, BN, BN), lambda t: (t, 0, 0)),
            pl.BlockSpec((1, BN, 1), lambda t: (t, 0, 0)),
            pl.BlockSpec((Din, Dh), lambda t: (0, 0)),
            pl.BlockSpec((1, Dh), lambda t: (0, 0)),
            pl.BlockSpec((Dh, 3 * Dh), lambda t: (0, 0)),
            pl.BlockSpec((Dh, 3 * Dh), lambda t: (0, 0)),
            pl.BlockSpec((1, 3 * Dh), lambda t: (0, 0)),
            pl.BlockSpec((1, 3 * Dh), lambda t: (0, 0)),
            pl.BlockSpec((Dh, Dout), lambda t: (0, 0)),
            pl.BlockSpec((1, Dout), lambda t: (0, 0)),
        ],
        out_specs=pl.BlockSpec((1, BN, Dout), lambda t: (t, 0, 0)),
        out_shape=jax.ShapeDtypeStruct((Tn, BN, Dout), jnp.float32),
        scratch_shapes=[pltpu.VMEM((BN, Dh), jnp.float32)],
    )(x, adj, mf, W_gcn, b_gcn.reshape(1, Dh), W_ih.T, W_hh.T,
      b_ih.reshape(1, 3 * Dh), b_hh.reshape(1, 3 * Dh), W_fc,
      b_fc.reshape(1, Dout))

    return jnp.transpose(seq.reshape(Tn, Bn, Nn, Dout), (1, 2, 0, 3))
